# baseline (speedup 1.0000x reference)
import functools

import jax
import jax.numpy as jnp
from jax import lax
from jax.experimental import pallas as pl
from jax.experimental.pallas import tpu as pltpu
from jax.experimental.pallas import tpu_sc as plsc

VOCAB = 100000
EMBED = 64
NFEAT = 26
HIDDEN = 1024
NCLASS = 2
BATCH = 4096

_NC = 2
_NS = 16
_NW = _NC * _NS

_ROWS = BATCH * NFEAT
_BPW = BATCH // _NW
_RPW = _ROWS // _NW
_BPC = 16
_NCHUNK = _BPW // _BPC
_CROWS = _BPC * NFEAT
_NPAIR = NFEAT // 2


def _gather_sc(idx, emb):
    mesh = plsc.VectorSubcoreMesh(core_axis_name="c", subcore_axis_name="s")

    @functools.partial(
        pl.kernel,
        mesh=mesh,
        compiler_params=pltpu.CompilerParams(use_tc_tiling_on_sc=False),
        out_type=jax.ShapeDtypeStruct((_NPAIR, BATCH, 2 * EMBED), jnp.float32),
        scratch_types=[
            pltpu.VMEM((_RPW,), jnp.int32),
            pltpu.VMEM((_CROWS, EMBED), jnp.float32),
            pltpu.VMEM((_CROWS, EMBED), jnp.float32),
            pltpu.VMEM((_NPAIR, _BPC, 2 * EMBED), jnp.float32),
            pltpu.SemaphoreType.DMA,
            pltpu.SemaphoreType.DMA,
        ],
    )
    def k(idx_hbm, emb_hbm, out_hbm, idx_v, rows_a, rows_b, tiled_v,
          sem_a, sem_b):
        wid = lax.axis_index("s") * _NC + lax.axis_index("c")
        b0 = wid * _BPW
        bufs = (rows_a, rows_b)
        sems = (sem_a, sem_b)
        pltpu.sync_copy(idx_hbm.at[pl.ds(b0 * NFEAT, _RPW)], idx_v)

        def fire(c):
            pltpu.async_copy(
                emb_hbm.at[idx_v.at[pl.ds(c * _CROWS, _CROWS)]],
                bufs[c % 2], sems[c % 2])

        def drain(c):
            pltpu.make_async_copy(
                emb_hbm.at[pl.ds(0, _CROWS)], bufs[c % 2], sems[c % 2]).wait()

        def rearrange(c):
            buf = bufs[c % 2]

            def body(bl, carry):
                for f in range(NFEAT):
                    t = f // 2
                    half = (f % 2) * EMBED
                    for kk in range(0, EMBED, 16):
                        tiled_v[t, bl, pl.ds(half + kk, 16)] = (
                            buf[bl * NFEAT + f, pl.ds(kk, 16)])
                return carry

            lax.fori_loop(0, _BPC, body, 0)

        fire(0)
        for c in range(_NCHUNK):
            drain(c)
            if c + 1 < _NCHUNK:
                fire(c + 1)
            rearrange(c)
            pltpu.sync_copy(
                tiled_v,
                out_hbm.at[pl.ds(0, _NPAIR), pl.ds(b0 + c * _BPC, _BPC)])

    return k(idx, emb)


_BB = 512


def _mlp_body(a_ref, w1_ref, b1_ref, w2_ref, b2_ref, o_ref):
    acc = jnp.zeros((_BB, HIDDEN), jnp.float32)
    for t in range(_NPAIR):
        acc += lax.dot_general(
            a_ref[t], w1_ref[:, t * 2 * EMBED:(t + 1) * 2 * EMBED],
            (((1,), (1,)), ((), ())), preferred_element_type=jnp.float32)
    h = jnp.maximum(acc + b1_ref[...], 0.0)
    o = lax.dot_general(h, w2_ref[...], (((1,), (1,)), ((), ())),
                        preferred_element_type=jnp.float32)
    o_ref[...] = o + b2_ref[...]


def _mlp_tc(a3, W1, b1, W2, b2):
    din = NFEAT * EMBED
    return pl.pallas_call(
        _mlp_body,
        grid=(BATCH // _BB,),
        in_specs=[
            pl.BlockSpec((_NPAIR, _BB, 2 * EMBED), lambda i: (0, i, 0)),
            pl.BlockSpec((HIDDEN, din), lambda i: (0, 0)),
            pl.BlockSpec((1, HIDDEN), lambda i: (0, 0)),
            pl.BlockSpec((NCLASS, HIDDEN), lambda i: (0, 0)),
            pl.BlockSpec((1, NCLASS), lambda i: (0, 0)),
        ],
        out_specs=pl.BlockSpec((_BB, NCLASS), lambda i: (i, 0)),
        out_shape=jax.ShapeDtypeStruct((BATCH, NCLASS), jnp.float32),
    )(a3, W1, b1.reshape(1, HIDDEN), W2, b2.reshape(1, NCLASS))


def kernel(x, emb, W1, b1, W2, b2):
    flat_idx = x.reshape(-1).astype(jnp.int32)
    a3 = _gather_sc(flat_idx, emb)
    return _mlp_tc(a3, W1, b1, W2, b2)


# speedup vs baseline: 1.1877x; 1.1877x over previous
import functools

import jax
import jax.numpy as jnp
from jax import lax
from jax.experimental import pallas as pl
from jax.experimental.pallas import tpu as pltpu
from jax.experimental.pallas import tpu_sc as plsc

VOCAB = 100000
EMBED = 64
NFEAT = 26
HIDDEN = 1024
NCLASS = 2
BATCH = 4096

_NC = 2
_NS = 16
_NW = _NC * _NS

_ROWS = BATCH * NFEAT
_RPW = _ROWS // _NW
_CHUNK = 1664
_NCHUNK = _RPW // _CHUNK


def _gather_sc(idx, emb):
    mesh = plsc.VectorSubcoreMesh(core_axis_name="c", subcore_axis_name="s")

    @functools.partial(
        pl.kernel,
        mesh=mesh,
        compiler_params=pltpu.CompilerParams(use_tc_tiling_on_sc=False),
        out_type=jax.ShapeDtypeStruct((_ROWS, EMBED), jnp.float32),
        scratch_types=[
            pltpu.VMEM((_CHUNK,), jnp.int32),
            pltpu.VMEM((_CHUNK, EMBED), jnp.float32),
            pltpu.SemaphoreType.DMA,
        ],
    )
    def k(idx_hbm, emb_hbm, out_hbm, idx_v, rows_v, sem):
        wid = lax.axis_index("s") * _NC + lax.axis_index("c")
        base = wid * _RPW
        for c in range(_NCHUNK):
            off = base + c * _CHUNK
            pltpu.sync_copy(idx_hbm.at[pl.ds(off, _CHUNK)], idx_v)
            pltpu.async_copy(emb_hbm.at[idx_v], rows_v, sem).wait()
            pltpu.sync_copy(rows_v, out_hbm.at[pl.ds(off, _CHUNK)])

    return k(idx, emb)


_BB = 1024


def _mlp_body(a_ref, w1_ref, b1_ref, w2_ref, b2_ref, o_ref):
    h = lax.dot_general(a_ref[...], w1_ref[...], (((1,), (1,)), ((), ())),
                        preferred_element_type=jnp.float32)
    h = jnp.maximum(h + b1_ref[...], 0.0)
    o = lax.dot_general(h, w2_ref[...], (((1,), (1,)), ((), ())),
                        preferred_element_type=jnp.float32)
    o_ref[...] = o + b2_ref[...]


def _mlp_tc(a, W1, b1, W2, b2):
    din = NFEAT * EMBED
    return pl.pallas_call(
        _mlp_body,
        grid=(BATCH // _BB,),
        in_specs=[
            pl.BlockSpec((_BB, din), lambda i: (i, 0)),
            pl.BlockSpec((HIDDEN, din), lambda i: (0, 0)),
            pl.BlockSpec((1, HIDDEN), lambda i: (0, 0)),
            pl.BlockSpec((NCLASS, HIDDEN), lambda i: (0, 0)),
            pl.BlockSpec((1, NCLASS), lambda i: (0, 0)),
        ],
        out_specs=pl.BlockSpec((_BB, NCLASS), lambda i: (i, 0)),
        out_shape=jax.ShapeDtypeStruct((BATCH, NCLASS), jnp.float32),
    )(a, W1, b1.reshape(1, HIDDEN), W2, b2.reshape(1, NCLASS))


def kernel(x, emb, W1, b1, W2, b2):
    flat_idx = x.reshape(-1).astype(jnp.int32)
    gathered = _gather_sc(flat_idx, emb)
    a = gathered.reshape(BATCH, NFEAT * EMBED)
    return _mlp_tc(a, W1, b1, W2, b2)
